# baseline (speedup 1.0000x reference)
import jax
import jax.numpy as jnp
from jax import lax
from jax.experimental import pallas as pl
from jax.experimental.pallas import tpu as pltpu


def _ceil_to(n, m):
    return ((n + m - 1) // m) * m


_DN_T = (((0,), (0,)), ((), ()))


def _mlp_t_kernel(a_emb_ref, a_head_ref,
                  xt_ref, w1_ref, b1_ref, w2_ref, b2_ref, fcw_ref, fcb_ref,
                  scores_t_ref, emb_t_ref):
    a_emb = a_emb_ref[0, 0]
    a_head = a_head_ref[0, 0]

    h = lax.dot_general(w1_ref[...], xt_ref[...], _DN_T,
                        preferred_element_type=jnp.float32)
    h = h + b1_ref[...].T
    h = jnp.maximum(h, 0.0) + a_emb * jnp.minimum(h, 0.0)

    emb = jnp.dot(w2_ref[...], h, preferred_element_type=jnp.float32)
    emb = emb + b2_ref[...].T
    emb_t_ref[...] = emb

    e = jnp.maximum(emb, 0.0) + a_head * jnp.minimum(emb, 0.0)
    z = lax.dot_general(e, fcw_ref[...], _DN_T,
                        preferred_element_type=jnp.float32)
    z = z + fcb_ref[...]
    m = jnp.max(z, axis=-1, keepdims=True)
    s = z - m
    lse = jnp.log(jnp.sum(jnp.exp(s), axis=-1, keepdims=True))
    scores_t_ref[...] = s - lse


def _head_only_kernel(scalars_ref, x_ref, w_ref, b_ref, out_ref):
    a = scalars_ref[0]
    x = x_ref[...]
    xa = jnp.maximum(x, 0.0) + a * jnp.minimum(x, 0.0)
    z = jnp.dot(xa, w_ref[...], preferred_element_type=jnp.float32) + b_ref[...]
    m = jnp.max(z, axis=-1, keepdims=True)
    s = z - m
    lse = jnp.log(jnp.sum(jnp.exp(s), axis=-1, keepdims=True))
    out_ref[...] = s - lse


def _aug_head(aug_sample, prelu_alpha, fc1_w_t, fc1_b):
    B, d = aug_sample.shape
    out_dim = fc1_w_t.shape[1]
    TB = min(1024, _ceil_to(B, 8))
    pad_B = _ceil_to(B, TB)
    aug = aug_sample.astype(jnp.float32)
    if pad_B != B:
        aug = jnp.pad(aug, ((0, pad_B - B), (0, 0)))
    scalars = jnp.reshape(prelu_alpha, (1,)).astype(jnp.float32)
    out = pl.pallas_call(
        _head_only_kernel,
        out_shape=jax.ShapeDtypeStruct((pad_B, out_dim), jnp.float32),
        grid_spec=pltpu.PrefetchScalarGridSpec(
            num_scalar_prefetch=1,
            grid=(pad_B // TB,),
            in_specs=[
                pl.BlockSpec((TB, d), lambda i, a: (i, 0)),
                pl.BlockSpec((d, out_dim), lambda i, a: (0, 0)),
                pl.BlockSpec((1, out_dim), lambda i, a: (0, 0)),
            ],
            out_specs=pl.BlockSpec((TB, out_dim), lambda i, a: (i, 0)),
        ),
        compiler_params=pltpu.CompilerParams(
            dimension_semantics=("parallel",)),
    )(scalars, aug, fc1_w_t.astype(jnp.float32), fc1_b.astype(jnp.float32))
    return out[:B]


def kernel(emb_w1_t, emb_b1, emb_prelu_alpha, emb_w2_t, emb_b2,
           prelu_alpha, fc1_w_t, fc1_b, x, aug_sample):
    if aug_sample.shape[0] != 0:
        return _aug_head(aug_sample, prelu_alpha, fc1_w_t, fc1_b)

    B = x.shape[0]
    d_in = 1
    for s in x.shape[1:]:
        d_in *= s
    xt = x.transpose(*range(1, x.ndim), 0).reshape(d_in, B)
    xt = xt.astype(jnp.float32)

    d_hidden = emb_w1_t.shape[1]
    h_pad = _ceil_to(d_hidden, 128)
    w1, b1 = emb_w1_t, emb_b1
    w2 = jnp.transpose(emb_w2_t)
    if h_pad != d_hidden:
        w1 = jnp.pad(w1, ((0, 0), (0, h_pad - d_hidden)))
        b1 = jnp.pad(b1, ((0, 0), (0, h_pad - d_hidden)))
        w2 = jnp.pad(w2, ((0, 0), (0, h_pad - d_hidden)))
    emb_dim = w2.shape[0]
    out_dim = fc1_w_t.shape[1]

    TB = 2048
    while TB > 8 and B % TB != 0:
        TB //= 2
    pad_B = _ceil_to(B, TB)
    if pad_B != B:
        xt = jnp.pad(xt, ((0, 0), (0, pad_B - B)))

    scores_t, emb_t = pl.pallas_call(
        _mlp_t_kernel,
        out_shape=(jax.ShapeDtypeStruct((pad_B, out_dim), jnp.float32),
                   jax.ShapeDtypeStruct((emb_dim, pad_B), jnp.float32)),
        grid=(pad_B // TB,),
        in_specs=[
            pl.BlockSpec((1, 1), lambda i: (0, 0)),
            pl.BlockSpec((1, 1), lambda i: (0, 0)),
            pl.BlockSpec((d_in, TB), lambda i: (0, i)),
            pl.BlockSpec((d_in, h_pad), lambda i: (0, 0)),
            pl.BlockSpec((1, h_pad), lambda i: (0, 0)),
            pl.BlockSpec((emb_dim, h_pad), lambda i: (0, 0)),
            pl.BlockSpec((1, emb_dim), lambda i: (0, 0)),
            pl.BlockSpec((emb_dim, out_dim), lambda i: (0, 0)),
            pl.BlockSpec((1, out_dim), lambda i: (0, 0)),
        ],
        out_specs=[
            pl.BlockSpec((TB, out_dim), lambda i: (i, 0)),
            pl.BlockSpec((emb_dim, TB), lambda i: (0, i)),
        ],
        compiler_params=pltpu.CompilerParams(
            dimension_semantics=("parallel",),
            vmem_limit_bytes=64 * 1024 * 1024,
        ),
    )(emb_prelu_alpha.astype(jnp.float32), prelu_alpha.astype(jnp.float32),
      xt, w1, b1, w2, emb_b2, fc1_w_t, fc1_b)

    scores = scores_t[:B]
    emb = emb_t[:, :B].T
    return scores, emb


# speedup vs baseline: 1.0267x; 1.0267x over previous
import jax
import jax.numpy as jnp
from jax import lax
from jax.experimental import pallas as pl
from jax.experimental.pallas import tpu as pltpu


def _ceil_to(n, m):
    return ((n + m - 1) // m) * m


_DN_T = (((0,), (0,)), ((), ()))


def _mlp_t_kernel(a_emb_ref, a_head_ref,
                  xt_ref, w1_ref, b1_ref, w2_ref, b2_ref, fcw_ref, fcb_ref,
                  scores_t_ref, emb_t_ref):
    a_emb = a_emb_ref[0, 0]
    a_head = a_head_ref[0, 0]

    h = lax.dot_general(w1_ref[...], xt_ref[...], _DN_T,
                        preferred_element_type=jnp.float32)
    h = h + b1_ref[...].T
    h = jnp.maximum(h, 0.0) + a_emb * jnp.minimum(h, 0.0)

    emb = jnp.dot(w2_ref[...], h, preferred_element_type=jnp.float32)
    emb = emb + b2_ref[...].T
    emb_t_ref[...] = emb

    e = jnp.maximum(emb, 0.0) + a_head * jnp.minimum(emb, 0.0)
    z = lax.dot_general(e, fcw_ref[...], _DN_T,
                        preferred_element_type=jnp.float32)
    z = z + fcb_ref[...]
    m = jnp.max(z, axis=-1, keepdims=True)
    s = z - m
    lse = jnp.log(jnp.sum(jnp.exp(s), axis=-1, keepdims=True))
    scores_t_ref[...] = s - lse


def _head_only_kernel(scalars_ref, x_ref, w_ref, b_ref, out_ref):
    a = scalars_ref[0]
    x = x_ref[...]
    xa = jnp.maximum(x, 0.0) + a * jnp.minimum(x, 0.0)
    z = jnp.dot(xa, w_ref[...], preferred_element_type=jnp.float32) + b_ref[...]
    m = jnp.max(z, axis=-1, keepdims=True)
    s = z - m
    lse = jnp.log(jnp.sum(jnp.exp(s), axis=-1, keepdims=True))
    out_ref[...] = s - lse


def _aug_head(aug_sample, prelu_alpha, fc1_w_t, fc1_b):
    B, d = aug_sample.shape
    out_dim = fc1_w_t.shape[1]
    TB = min(1024, _ceil_to(B, 8))
    pad_B = _ceil_to(B, TB)
    aug = aug_sample.astype(jnp.float32)
    if pad_B != B:
        aug = jnp.pad(aug, ((0, pad_B - B), (0, 0)))
    scalars = jnp.reshape(prelu_alpha, (1,)).astype(jnp.float32)
    out = pl.pallas_call(
        _head_only_kernel,
        out_shape=jax.ShapeDtypeStruct((pad_B, out_dim), jnp.float32),
        grid_spec=pltpu.PrefetchScalarGridSpec(
            num_scalar_prefetch=1,
            grid=(pad_B // TB,),
            in_specs=[
                pl.BlockSpec((TB, d), lambda i, a: (i, 0)),
                pl.BlockSpec((d, out_dim), lambda i, a: (0, 0)),
                pl.BlockSpec((1, out_dim), lambda i, a: (0, 0)),
            ],
            out_specs=pl.BlockSpec((TB, out_dim), lambda i, a: (i, 0)),
        ),
        compiler_params=pltpu.CompilerParams(
            dimension_semantics=("parallel",)),
    )(scalars, aug, fc1_w_t.astype(jnp.float32), fc1_b.astype(jnp.float32))
    return out[:B]


def kernel(emb_w1_t, emb_b1, emb_prelu_alpha, emb_w2_t, emb_b2,
           prelu_alpha, fc1_w_t, fc1_b, x, aug_sample):
    if aug_sample.shape[0] != 0:
        return _aug_head(aug_sample, prelu_alpha, fc1_w_t, fc1_b)

    B = x.shape[0]
    d_in = 1
    for s in x.shape[1:]:
        d_in *= s
    xt = x.transpose(*range(1, x.ndim), 0).reshape(d_in, B)
    xt = xt.astype(jnp.float32)

    d_hidden = emb_w1_t.shape[1]
    h_pad = _ceil_to(d_hidden, 128)
    w1, b1 = emb_w1_t, emb_b1
    w2 = jnp.transpose(emb_w2_t)
    if h_pad != d_hidden:
        w1 = jnp.pad(w1, ((0, 0), (0, h_pad - d_hidden)))
        b1 = jnp.pad(b1, ((0, 0), (0, h_pad - d_hidden)))
        w2 = jnp.pad(w2, ((0, 0), (0, h_pad - d_hidden)))
    emb_dim = w2.shape[0]
    out_dim = fc1_w_t.shape[1]

    TB = 4096
    while TB > 8 and B % TB != 0:
        TB //= 2
    pad_B = _ceil_to(B, TB)
    if pad_B != B:
        xt = jnp.pad(xt, ((0, 0), (0, pad_B - B)))

    scores_t, emb_t = pl.pallas_call(
        _mlp_t_kernel,
        out_shape=(jax.ShapeDtypeStruct((pad_B, out_dim), jnp.float32),
                   jax.ShapeDtypeStruct((emb_dim, pad_B), jnp.float32)),
        grid=(pad_B // TB,),
        in_specs=[
            pl.BlockSpec((1, 1), lambda i: (0, 0)),
            pl.BlockSpec((1, 1), lambda i: (0, 0)),
            pl.BlockSpec((d_in, TB), lambda i: (0, i)),
            pl.BlockSpec((d_in, h_pad), lambda i: (0, 0)),
            pl.BlockSpec((1, h_pad), lambda i: (0, 0)),
            pl.BlockSpec((emb_dim, h_pad), lambda i: (0, 0)),
            pl.BlockSpec((1, emb_dim), lambda i: (0, 0)),
            pl.BlockSpec((emb_dim, out_dim), lambda i: (0, 0)),
            pl.BlockSpec((1, out_dim), lambda i: (0, 0)),
        ],
        out_specs=[
            pl.BlockSpec((TB, out_dim), lambda i: (i, 0)),
            pl.BlockSpec((emb_dim, TB), lambda i: (0, i)),
        ],
        compiler_params=pltpu.CompilerParams(
            dimension_semantics=("parallel",),
            vmem_limit_bytes=64 * 1024 * 1024,
        ),
    )(emb_prelu_alpha.astype(jnp.float32), prelu_alpha.astype(jnp.float32),
      xt, w1, b1, w2, emb_b2, fc1_w_t, fc1_b)

    scores = scores_t[:B]
    emb = emb_t[:, :B].T
    return scores, emb
